# Initial kernel scaffold; baseline (speedup 1.0000x reference)
import jax
import jax.numpy as jnp
from jax.experimental import pallas as pl


def kernel(features_list, biases_mat_list, batch_node_list, adjs, n_ids, device, RL_thresholds, W1, b1, W2, b2, Wf, bf, a1, a2, Wm, bm, w_omega, b_omega, u_omega):
    raise NotImplementedError("write your pallas kernel here")


# speedup vs baseline: 2.8431x; 2.8431x over previous
import functools

import jax
import jax.numpy as jnp
from jax import lax
from jax.experimental import pallas as pl
from jax.experimental.pallas import tpu as pltpu

P = 3
NBATCH = 1024
F = 128
NH = 8
HEAD_IN = F // NH
OUT_DIM = 64
OUT_SZ = OUT_DIM // NH
HID = 128


def _dense_body(xg_ref, A_ref, bias_ref, W1_ref, b1_ref, W2_ref, b2_ref,
                Wf_ref, bf_ref, a1_ref, a2_ref, Wm_ref, bm_ref,
                wom_ref, bom_ref, uom_ref,
                out_ref, multi_scr):
    i = pl.program_id(0)
    x = xg_ref[0]
    A = A_ref[0]
    bias = bias_ref[0]

    deg = jnp.maximum(jnp.sum(A, axis=1, keepdims=True), 1.0)
    agg1 = jnp.dot(A, x, preferred_element_type=jnp.float32) / deg
    h = jax.nn.relu(jnp.dot(agg1, W1_ref[0], preferred_element_type=jnp.float32)
                    + b1_ref[0])
    agg2 = jnp.dot(A, h, preferred_element_type=jnp.float32) / deg
    fe = jnp.dot(agg2, W2_ref[0], preferred_element_type=jnp.float32) + b2_ref[0]

    attns = []
    for nh in range(NH):
        xh = fe[:, nh * HEAD_IN:(nh + 1) * HEAD_IN]
        f = jnp.dot(xh, Wf_ref[0, nh], preferred_element_type=jnp.float32) \
            + bf_ref[0, nh]
        f1 = jnp.dot(f, a1_ref[0, nh].reshape(OUT_SZ, 1),
                     preferred_element_type=jnp.float32)
        f2 = jnp.dot(f, a2_ref[0, nh].reshape(OUT_SZ, 1),
                     preferred_element_type=jnp.float32)
        logits = f1 + f2.T
        z = jnp.where(logits >= 0.0, logits, 0.2 * logits) + bias
        m = jnp.max(z, axis=1, keepdims=True)
        e = jnp.exp(z - m)
        s = jnp.sum(e, axis=1, keepdims=True)
        coefs = e / s
        av = jnp.dot(coefs, f, preferred_element_type=jnp.float32)
        attns.append(jnp.where(av > 0.0, av, jnp.exp(av) - 1.0))
    h_1 = jnp.concatenate(attns, axis=-1)
    h1t = jnp.dot(h_1, Wm_ref[...], preferred_element_type=jnp.float32) \
        + bm_ref[...]
    multi_scr[pl.ds(i, 1)] = h1t[None]

    @pl.when(i == P - 1)
    def _():
        ms = [multi_scr[j] for j in range(P)]
        vus = []
        for j in range(P):
            v = jnp.tanh(jnp.dot(ms[j], wom_ref[...],
                                 preferred_element_type=jnp.float32)
                         + bom_ref[...])
            vu = jnp.dot(v, uom_ref[...].reshape(HID, 1),
                         preferred_element_type=jnp.float32)
            vus.append(vu)
        vu_all = jnp.concatenate(vus, axis=-1)
        mx = jnp.max(vu_all, axis=1, keepdims=True)
        ev = jnp.exp(vu_all - mx)
        al = ev / jnp.sum(ev, axis=1, keepdims=True)
        acc = al[:, 0:1] * ms[0]
        for j in range(1, P):
            acc = acc + al[:, j:j + 1] * ms[j]
        out_ref[...] = acc


@jax.jit
def _dense_call(xg, A, bias, W1, b1, W2, b2, Wf, bf, a1, a2, Wm, bm,
                w_omega, b_omega, u_omega):
    grid = (P,)
    bs_meta3 = lambda shp: pl.BlockSpec((1,) + shp, lambda i: (i,) + (0,) * len(shp))
    bs_full = lambda shp: pl.BlockSpec(shp, lambda i: (0,) * len(shp))
    return pl.pallas_call(
        _dense_body,
        grid=grid,
        in_specs=[
            bs_meta3((NBATCH, F)),
            bs_meta3((NBATCH, NBATCH)),
            bs_meta3((NBATCH, NBATCH)),
            bs_meta3((F, HID)),
            bs_meta3((1, HID)),
            bs_meta3((HID, F)),
            bs_meta3((1, F)),
            bs_meta3((NH, HEAD_IN, OUT_SZ)),
            bs_meta3((NH, 1, OUT_SZ)),
            bs_meta3((NH, OUT_SZ)),
            bs_meta3((NH, OUT_SZ)),
            bs_full((OUT_DIM, OUT_DIM)),
            bs_full((1, OUT_DIM)),
            bs_full((OUT_DIM, HID)),
            bs_full((1, HID)),
            bs_full((1, HID)),
        ],
        out_specs=pl.BlockSpec((NBATCH, OUT_DIM), lambda i: (0, 0)),
        out_shape=jax.ShapeDtypeStruct((NBATCH, OUT_DIM), jnp.float32),
        scratch_shapes=[pltpu.VMEM((P, NBATCH, OUT_DIM), jnp.float32)],
    )(xg, A, bias, W1, b1[:, None, :], W2, b2[:, None, :], Wf,
      bf[:, :, None, :], a1, a2, Wm, bm[None], w_omega, b_omega[None],
      u_omega[None])


def kernel(features_list, biases_mat_list, batch_node_list, adjs, n_ids,
           device, RL_thresholds, W1, b1, W2, b2, Wf, bf, a1, a2, Wm, bm,
           w_omega, b_omega, u_omega):
    xg = jnp.take_along_axis(features_list, n_ids[:, :, None], axis=1)
    ones = jnp.ones((P, adjs.shape[2]), jnp.float32)
    A = jnp.zeros((P, NBATCH, NBATCH), jnp.float32)
    pidx = jnp.broadcast_to(jnp.arange(P)[:, None], adjs[:, 1].shape)
    A = A.at[pidx, adjs[:, 1], adjs[:, 0]].add(ones)
    rows = jnp.take_along_axis(biases_mat_list, batch_node_list[:, :, None],
                               axis=1)
    bias = jnp.take_along_axis(rows, batch_node_list[:, None, :], axis=2)
    return _dense_call(xg, A, bias, W1, b1, W2, b2, Wf, bf, a1, a2,
                       Wm, bm, w_omega, b_omega, u_omega)
